# baseline (speedup 1.0000x reference)
import functools

import jax
import jax.numpy as jnp
from jax import lax
from jax.experimental import pallas as pl
from jax.experimental.pallas import tpu as pltpu
from jax.experimental.pallas import tpu_sc as plsc

N = 10000
E = 320000
D = 128

NC = 2
NS = 16
NW = NC * NS
EW = E // NW
CH = 40
NCHUNK = EW // CH
K = 5
G = 25
NGRP = NCHUNK // G
GE = G * CH
WB = 40
NWB_TOT = N // WB
NWB = -(-NWB_TOT // NS)


def _attn_body(x_ref, w_ref, b_ref, o_ref):
    x = x_ref[...]
    z = lax.dot_general(x, w_ref[...], (((1,), (1,)), ((), ())),
                        preferred_element_type=jnp.float32)
    o_ref[...] = x * jax.nn.sigmoid(z + b_ref[...])


def _attended(x, W, b):
    blk = 1000
    return pl.pallas_call(
        _attn_body,
        grid=(N // blk,),
        in_specs=[
            pl.BlockSpec((blk, D), lambda i: (i, 0)),
            pl.BlockSpec((D, D), lambda i: (0, 0)),
            pl.BlockSpec((1, D), lambda i: (0, 0)),
        ],
        out_specs=pl.BlockSpec((blk, D), lambda i: (i, 0)),
        out_shape=jax.ShapeDtypeStruct((N, D), jnp.float32),
    )(x, W, b.reshape(1, D))


def _sc_body(att, row5, col5, w, parts, acc, colg, rowg, wg, rows_v,
             sbuf, psem, zsem, gsem, ssem):
    cid = lax.axis_index("c")
    sid = lax.axis_index("s")
    wid = cid * NS + sid
    base = wid * EW

    def fire_idx_prefetch(g, gb):
        pltpu.async_copy(col5.at[wid, g], colg.at[gb], psem)
        pltpu.async_copy(row5.at[wid, g], rowg.at[gb], psem)
        pltpu.async_copy(w.at[pl.ds(base + g * GE, GE)], wg.at[gb, 0], psem)

    def wait_idx_prefetch(g, gb):
        pltpu.make_async_copy(col5.at[wid, g], colg.at[gb], psem).wait()
        pltpu.make_async_copy(row5.at[wid, g], rowg.at[gb], psem).wait()
        pltpu.make_async_copy(w.at[pl.ds(base + g * GE, GE)],
                              wg.at[gb, 0], psem).wait()

    fire_idx_prefetch(0, 0)

    def zrow(r, _):
        def zcol(u, _):
            sbuf[0, r, pl.ds(u * 16, 16)] = jnp.zeros((16,), jnp.float32)
            return 0
        return lax.fori_loop(0, D // 16, zcol, 0)
    lax.fori_loop(0, WB, zrow, 0)

    for k in range(NWB):
        c = k * NS + sid

        @pl.when(c < NWB_TOT)
        def _():
            pltpu.async_copy(sbuf.at[0], acc.at[pl.ds(c * WB, WB)], zsem)
    for k in range(NWB):
        c = k * NS + sid

        @pl.when(c < NWB_TOT)
        def _():
            pltpu.make_async_copy(sbuf.at[0], acc.at[pl.ds(c * WB, WB)],
                                  zsem).wait()
    plsc.subcore_barrier()

    def group(g, _):
        gb = g % 2
        gb1 = (g + 1) % 2

        @pl.when(g == 0)
        def _():
            wait_idx_prefetch(0, 0)

        def fire_gather(t, slot):
            pltpu.async_copy(att.at[colg.at[gb, t, 0]],
                             rows_v.at[slot], gsem.at[slot])

        @pl.when(g == 0)
        def _():
            for t in range(3):
                fire_gather(t, t)

        for t in range(G):
            slot = t % 3
            sb = t % 2
            pltpu.make_async_copy(att.at[colg.at[gb, t, 0]],
                                  rows_v.at[slot], gsem.at[slot]).wait()

            def drain_sb():
                pltpu.make_async_copy(sbuf.at[sb],
                                      acc.at[rowg.at[gb, t, 0]],
                                      ssem.at[sb]).wait()
            if t >= 2:
                drain_sb()
            else:
                @pl.when(g > 0)
                def _():
                    drain_sb()

            if t == 2:
                @pl.when(g + 1 < NGRP)
                def _():
                    fire_idx_prefetch(g + 1, gb1)

            @plsc.parallel_loop(0, CH, unroll=4)
            def _(e):
                wv = plsc.load_gather(
                    wg.at[gb, 0], [jnp.full((16,), t * CH, jnp.int32) + e])
                for u in range(D // 16):
                    sl = pl.ds(u * 16, 16)
                    sbuf[sb, e, sl] = rows_v[slot, e, sl] * wv

            pltpu.async_copy(sbuf.at[sb], acc.at[rowg.at[gb, t, 0]],
                             ssem.at[sb], add=True)

            if t + 3 < G:
                fire_gather(t + 3, slot)
            else:
                if t == G - 3:
                    @pl.when(g + 1 < NGRP)
                    def _():
                        wait_idx_prefetch(g + 1, gb1)

                @pl.when(g + 1 < NGRP)
                def _():
                    pltpu.async_copy(att.at[colg.at[gb1, slot, 0]],
                                     rows_v.at[slot], gsem.at[slot])
        return 0
    lax.fori_loop(0, NGRP, group, 0)

    for sb in range(2):
        pltpu.make_async_copy(sbuf.at[sb], acc.at[rowg.at[0, 0, 0]],
                              ssem.at[sb]).wait()
    plsc.subcore_barrier()
    for k in range(NWB):
        c = k * NS + sid
        st = k % 2

        @pl.when(c < NWB_TOT)
        def _():
            sl = pl.ds(c * WB, WB)
            if k >= 2:
                pltpu.make_async_copy(sbuf.at[st], parts.at[cid, sl],
                                      ssem.at[st]).wait()
            pltpu.sync_copy(acc.at[sl], sbuf.at[st])
            pltpu.async_copy(sbuf.at[st], parts.at[cid, sl], ssem.at[st])
    for k in range(max(0, NWB - 2), NWB):
        c = k * NS + sid

        @pl.when(c < NWB_TOT)
        def _():
            sl = pl.ds(c * WB, WB)
            pltpu.make_async_copy(sbuf.at[k % 2], parts.at[cid, sl],
                                  ssem.at[k % 2]).wait()


def _sc_scatter(att, row5, col5, w):
    mesh = plsc.VectorSubcoreMesh(core_axis_name="c", subcore_axis_name="s",
                                  num_cores=NC, num_subcores=NS)
    f = pl.kernel(
        _sc_body,
        out_type=jax.ShapeDtypeStruct((NC, N, D), jnp.float32),
        mesh=mesh,
        scratch_types=[
            pltpu.VMEM_SHARED((N, D), jnp.float32),
            pltpu.VMEM((2, G, 1, CH), jnp.int32),
            pltpu.VMEM((2, G, 1, CH), jnp.int32),
            pltpu.VMEM((2, 1, GE), jnp.float32),
            pltpu.VMEM((3, CH, D), jnp.float32),
            pltpu.VMEM((2, CH, D), jnp.float32),
            pltpu.SemaphoreType.DMA,
            pltpu.SemaphoreType.DMA,
            pltpu.SemaphoreType.DMA((3,)),
            pltpu.SemaphoreType.DMA((2,)),
        ],
        compiler_params=pltpu.CompilerParams(needs_layout_passes=False),
    )
    return f(att, row5, col5, w)


def _add_body(p_ref, o_ref):
    o_ref[...] = p_ref[0] + p_ref[1]


def _combine(parts):
    blk = 1000
    return pl.pallas_call(
        _add_body,
        grid=(N // blk,),
        in_specs=[pl.BlockSpec((NC, blk, D), lambda i: (0, i, 0))],
        out_specs=pl.BlockSpec((blk, D), lambda i: (i, 0)),
        out_shape=jax.ShapeDtypeStruct((N, D), jnp.float32),
    )(parts)


def kernel(x, edge_index, edge_weight, W, b):
    row = edge_index[0].astype(jnp.int32)
    col = edge_index[1].astype(jnp.int32)
    row5 = row.reshape(NW, NGRP, G, 1, CH)
    col5 = col.reshape(NW, NGRP, G, 1, CH)
    att = _attended(x, W, b)
    parts = _sc_scatter(att, row5, col5, edge_weight)
    return _combine(parts)


# speedup vs baseline: 1.0033x; 1.0033x over previous
import functools

import jax
import jax.numpy as jnp
from jax import lax
from jax.experimental import pallas as pl
from jax.experimental.pallas import tpu as pltpu
from jax.experimental.pallas import tpu_sc as plsc

N = 10000
E = 320000
D = 128

NC = 2
NS = 16
NW = NC * NS
EW = E // NW
CH = 40
NCHUNK = EW // CH
K = 5
G = 25
NGRP = NCHUNK // G
GE = G * CH
WB = 40
NWB_TOT = N // WB
NWB = -(-NWB_TOT // NS)


def _attn_body(x_ref, w_ref, b_ref, o_ref):
    x = x_ref[...]
    z = lax.dot_general(x, w_ref[...], (((1,), (1,)), ((), ())),
                        preferred_element_type=jnp.float32)
    o_ref[...] = x * jax.nn.sigmoid(z + b_ref[...])


def _attended(x, W, b):
    blk = 1000
    return pl.pallas_call(
        _attn_body,
        grid=(N // blk,),
        in_specs=[
            pl.BlockSpec((blk, D), lambda i: (i, 0)),
            pl.BlockSpec((D, D), lambda i: (0, 0)),
            pl.BlockSpec((1, D), lambda i: (0, 0)),
        ],
        out_specs=pl.BlockSpec((blk, D), lambda i: (i, 0)),
        out_shape=jax.ShapeDtypeStruct((N, D), jnp.float32),
    )(x, W, b.reshape(1, D))


def _sc_body(att, row5, col5, w, parts, acc, colg, rowg, wg, rows_v,
             sbuf, psem, zsem, gsem, ssem):
    cid = lax.axis_index("c")
    sid = lax.axis_index("s")
    wid = cid * NS + sid
    base = wid * EW

    def fire_idx_prefetch(g, gb):
        pltpu.async_copy(col5.at[wid, g], colg.at[gb], psem)
        pltpu.async_copy(row5.at[wid, g], rowg.at[gb], psem)
        pltpu.async_copy(w.at[pl.ds(base + g * GE, GE)], wg.at[gb, 0], psem)

    def wait_idx_prefetch(g, gb):
        pltpu.make_async_copy(col5.at[wid, g], colg.at[gb], psem).wait()
        pltpu.make_async_copy(row5.at[wid, g], rowg.at[gb], psem).wait()
        pltpu.make_async_copy(w.at[pl.ds(base + g * GE, GE)],
                              wg.at[gb, 0], psem).wait()

    fire_idx_prefetch(0, 0)

    def zrow(r, _):
        def zcol(u, _):
            sbuf[0, r, pl.ds(u * 16, 16)] = jnp.zeros((16,), jnp.float32)
            return 0
        return lax.fori_loop(0, D // 16, zcol, 0)
    lax.fori_loop(0, WB, zrow, 0)

    for k in range(NWB):
        c = k * NS + sid

        @pl.when(c < NWB_TOT)
        def _():
            pltpu.async_copy(sbuf.at[0], acc.at[pl.ds(c * WB, WB)], zsem)
    wait_idx_prefetch(0, 0)
    for t in range(3):
        pltpu.async_copy(att.at[colg.at[0, t, 0]], rows_v.at[t], gsem.at[t])
    for k in range(NWB):
        c = k * NS + sid

        @pl.when(c < NWB_TOT)
        def _():
            pltpu.make_async_copy(sbuf.at[0], acc.at[pl.ds(c * WB, WB)],
                                  zsem).wait()
    plsc.subcore_barrier()

    def group(g, _):
        gb = g % 2
        gb1 = (g + 1) % 2

        def fire_gather(t, slot):
            pltpu.async_copy(att.at[colg.at[gb, t, 0]],
                             rows_v.at[slot], gsem.at[slot])

        for t in range(G):
            slot = t % 3
            sb = t % 2
            pltpu.make_async_copy(att.at[colg.at[gb, t, 0]],
                                  rows_v.at[slot], gsem.at[slot]).wait()

            def drain_sb():
                pltpu.make_async_copy(sbuf.at[sb],
                                      acc.at[rowg.at[gb, t, 0]],
                                      ssem.at[sb]).wait()
            if t >= 2:
                drain_sb()
            else:
                @pl.when(g > 0)
                def _():
                    drain_sb()

            if t == 2:
                @pl.when(g + 1 < NGRP)
                def _():
                    fire_idx_prefetch(g + 1, gb1)

            @plsc.parallel_loop(0, CH, unroll=4)
            def _(e):
                wv = plsc.load_gather(
                    wg.at[gb, 0], [jnp.full((16,), t * CH, jnp.int32) + e])
                for u in range(D // 16):
                    sl = pl.ds(u * 16, 16)
                    sbuf[sb, e, sl] = rows_v[slot, e, sl] * wv

            pltpu.async_copy(sbuf.at[sb], acc.at[rowg.at[gb, t, 0]],
                             ssem.at[sb], add=True)

            if t + 3 < G:
                fire_gather(t + 3, slot)
            else:
                if t == G - 3:
                    @pl.when(g + 1 < NGRP)
                    def _():
                        wait_idx_prefetch(g + 1, gb1)

                @pl.when(g + 1 < NGRP)
                def _():
                    pltpu.async_copy(att.at[colg.at[gb1, slot, 0]],
                                     rows_v.at[slot], gsem.at[slot])
        return 0
    lax.fori_loop(0, NGRP, group, 0)

    for sb in range(2):
        pltpu.make_async_copy(sbuf.at[sb], acc.at[rowg.at[0, 0, 0]],
                              ssem.at[sb]).wait()
    plsc.subcore_barrier()
    for k in range(NWB):
        c = k * NS + sid
        st = k % 2

        @pl.when(c < NWB_TOT)
        def _():
            sl = pl.ds(c * WB, WB)
            if k >= 2:
                pltpu.make_async_copy(sbuf.at[st], parts.at[cid, sl],
                                      ssem.at[st]).wait()
            pltpu.sync_copy(acc.at[sl], sbuf.at[st])
            pltpu.async_copy(sbuf.at[st], parts.at[cid, sl], ssem.at[st])
    for k in range(max(0, NWB - 2), NWB):
        c = k * NS + sid

        @pl.when(c < NWB_TOT)
        def _():
            sl = pl.ds(c * WB, WB)
            pltpu.make_async_copy(sbuf.at[k % 2], parts.at[cid, sl],
                                  ssem.at[k % 2]).wait()


def _sc_scatter(att, row5, col5, w):
    mesh = plsc.VectorSubcoreMesh(core_axis_name="c", subcore_axis_name="s",
                                  num_cores=NC, num_subcores=NS)
    f = pl.kernel(
        _sc_body,
        out_type=jax.ShapeDtypeStruct((NC, N, D), jnp.float32),
        mesh=mesh,
        scratch_types=[
            pltpu.VMEM_SHARED((N, D), jnp.float32),
            pltpu.VMEM((2, G, 1, CH), jnp.int32),
            pltpu.VMEM((2, G, 1, CH), jnp.int32),
            pltpu.VMEM((2, 1, GE), jnp.float32),
            pltpu.VMEM((3, CH, D), jnp.float32),
            pltpu.VMEM((2, CH, D), jnp.float32),
            pltpu.SemaphoreType.DMA,
            pltpu.SemaphoreType.DMA,
            pltpu.SemaphoreType.DMA((3,)),
            pltpu.SemaphoreType.DMA((2,)),
        ],
        compiler_params=pltpu.CompilerParams(needs_layout_passes=False),
    )
    return f(att, row5, col5, w)


def _add_body(p_ref, o_ref):
    o_ref[...] = p_ref[0] + p_ref[1]


def _combine(parts):
    blk = 1000
    return pl.pallas_call(
        _add_body,
        grid=(N // blk,),
        in_specs=[pl.BlockSpec((NC, blk, D), lambda i: (0, i, 0))],
        out_specs=pl.BlockSpec((blk, D), lambda i: (i, 0)),
        out_shape=jax.ShapeDtypeStruct((N, D), jnp.float32),
    )(parts)


def kernel(x, edge_index, edge_weight, W, b):
    row = edge_index[0].astype(jnp.int32)
    col = edge_index[1].astype(jnp.int32)
    row5 = row.reshape(NW, NGRP, G, 1, CH)
    col5 = col.reshape(NW, NGRP, G, 1, CH)
    att = _attended(x, W, b)
    parts = _sc_scatter(att, row5, col5, edge_weight)
    return _combine(parts)
